# initial kernel scaffold (unmeasured)
import jax
import jax.numpy as jnp
from jax import lax
from jax.experimental import pallas as pl
from jax.experimental.pallas import tpu as pltpu

N_DEV = 4

_DID = getattr(pl, "DeviceIdType", None) or getattr(pltpu, "DeviceIdType")
_MESH = _DID.MESH

_CompilerParams = getattr(pltpu, "CompilerParams", None) or getattr(
    pltpu, "TPUCompilerParams"
)


def kernel(x, w_mat):
    m_total, k_per = x.shape
    _, n = w_mat.shape
    m_per = m_total // N_DEV

    def body(
        x_ref,
        w_ref,
        out_ref,
        send_buf,
        recv_buf,
        send_sems,
        recv_sems,
        amax_send,
        amax_recv,
        amax_send_sems,
        amax_recv_sems,
    ):
        my = lax.axis_index("i")
        right = lax.rem(my + 1, N_DEV)
        left = lax.rem(my + N_DEV - 1, N_DEV)

        barrier = pltpu.get_barrier_semaphore()
        for nbr in (left, right):
            pl.semaphore_signal(
                barrier, inc=1, device_id=(nbr,), device_id_type=_MESH
            )
        pl.semaphore_wait(barrier, 2)

        STRIP = 256

        def add_partial(dst, c, init):
            for r in range(0, m_per, STRIP):
                xs = x_ref[pl.ds(c * m_per + r, STRIP), :]
                p = jnp.dot(
                    xs,
                    w_ref[...],
                    preferred_element_type=jnp.float32,
                    precision=lax.Precision.HIGHEST,
                )
                if init:
                    dst[pl.ds(r, STRIP), :] = p
                else:
                    dst[pl.ds(r, STRIP), :] = dst[pl.ds(r, STRIP), :] + p

        add_partial(send_buf, left, True)
        rdma = pltpu.make_async_remote_copy(
            src_ref=send_buf,
            dst_ref=recv_buf.at[0],
            send_sem=send_sems.at[0],
            recv_sem=recv_sems.at[0],
            device_id=(right,),
            device_id_type=_MESH,
        )
        rdma.start()
        rdma.wait()

        for s in (1, 2):
            c = lax.rem(my - 1 - s + 2 * N_DEV, N_DEV)
            add_partial(recv_buf.at[s - 1], c, False)
            rdma = pltpu.make_async_remote_copy(
                src_ref=recv_buf.at[s - 1],
                dst_ref=recv_buf.at[s],
                send_sem=send_sems.at[s],
                recv_sem=recv_sems.at[s],
                device_id=(right,),
                device_id_type=_MESH,
            )
            rdma.start()
            rdma.wait()

        add_partial(recv_buf.at[2], my, False)

        local_amax = jnp.max(jnp.abs(recv_buf[2]))
        amax_send[...] = jnp.full((8, 128), local_amax, jnp.float32)
        descs = []
        for j in (1, 2, 3):
            peer = lax.rem(my + j, N_DEV)
            d = pltpu.make_async_remote_copy(
                src_ref=amax_send,
                dst_ref=amax_recv.at[j - 1],
                send_sem=amax_send_sems.at[j - 1],
                recv_sem=amax_recv_sems.at[j - 1],
                device_id=(peer,),
                device_id_type=_MESH,
            )
            d.start()
            descs.append(d)
        for d in descs:
            d.wait()
        gmax = local_amax
        for j in range(3):
            gmax = jnp.maximum(gmax, amax_recv[j, 0, 0])

        scale = gmax / 127.0
        q = jnp.clip(jnp.round(recv_buf[2] / scale), -127.0, 127.0)
        out_ref[...] = q.astype(jnp.int8)

    return pl.pallas_call(
        body,
        out_shape=jax.ShapeDtypeStruct((m_per, n), jnp.int8),
        in_specs=[
            pl.BlockSpec(memory_space=pltpu.VMEM),
            pl.BlockSpec(memory_space=pltpu.VMEM),
        ],
        out_specs=pl.BlockSpec(memory_space=pltpu.VMEM),
        scratch_shapes=[
            pltpu.VMEM((m_per, n), jnp.float32),
            pltpu.VMEM((3, m_per, n), jnp.float32),
            pltpu.SemaphoreType.DMA((3,)),
            pltpu.SemaphoreType.DMA((3,)),
            pltpu.VMEM((8, 128), jnp.float32),
            pltpu.VMEM((3, 8, 128), jnp.float32),
            pltpu.SemaphoreType.DMA((3,)),
            pltpu.SemaphoreType.DMA((3,)),
        ],
        compiler_params=_CompilerParams(collective_id=0),
    )(x, w_mat)


# baseline (device time: 422060 ns/iter reference)
import jax
import jax.numpy as jnp
from jax import lax
from jax.experimental import pallas as pl
from jax.experimental.pallas import tpu as pltpu

N_DEV = 4

_DID = getattr(pl, "DeviceIdType", None) or getattr(pltpu, "DeviceIdType")
_MESH = _DID.MESH

_CompilerParams = getattr(pltpu, "CompilerParams", None) or getattr(
    pltpu, "TPUCompilerParams"
)


def kernel(x, w_mat):
    m_total, k_per = x.shape
    _, n = w_mat.shape
    m_per = m_total // N_DEV

    def body(
        x_ref,
        w_ref,
        out_ref,
        x_chunk,
        send_buf,
        recv_buf,
        load_sem,
        send_sems,
        recv_sems,
        amax_send,
        amax_recv,
        amax_send_sems,
        amax_recv_sems,
    ):
        my = lax.axis_index("i")
        right = lax.rem(my + 1, N_DEV)
        left = lax.rem(my + N_DEV - 1, N_DEV)

        barrier = pltpu.get_barrier_semaphore()
        for nbr in (left, right):
            pl.semaphore_signal(
                barrier, inc=1, device_id=(nbr,), device_id_type=_MESH
            )
        pl.semaphore_wait(barrier, 2)

        STRIP = 256

        def load_chunk(c):
            cp = pltpu.make_async_copy(
                x_ref.at[pl.ds(c * m_per, m_per), :], x_chunk, load_sem
            )
            cp.start()
            cp.wait()

        def add_partial(dst, init):
            for r in range(0, m_per, STRIP):
                p = jnp.dot(
                    x_chunk[pl.ds(r, STRIP), :],
                    w_ref[...],
                    preferred_element_type=jnp.float32,
                    precision=lax.Precision.HIGHEST,
                )
                if init:
                    dst[pl.ds(r, STRIP), :] = p
                else:
                    dst[pl.ds(r, STRIP), :] = dst[pl.ds(r, STRIP), :] + p

        load_chunk(left)
        add_partial(send_buf, True)
        rdma = pltpu.make_async_remote_copy(
            src_ref=send_buf,
            dst_ref=recv_buf.at[0],
            send_sem=send_sems.at[0],
            recv_sem=recv_sems.at[0],
            device_id=(right,),
            device_id_type=_MESH,
        )
        rdma.start()
        rdma.wait()

        for s in (1, 2):
            c = lax.rem(my - 1 - s + 2 * N_DEV, N_DEV)
            load_chunk(c)
            add_partial(recv_buf.at[s - 1], False)
            rdma = pltpu.make_async_remote_copy(
                src_ref=recv_buf.at[s - 1],
                dst_ref=recv_buf.at[s],
                send_sem=send_sems.at[s],
                recv_sem=recv_sems.at[s],
                device_id=(right,),
                device_id_type=_MESH,
            )
            rdma.start()
            rdma.wait()

        load_chunk(my)
        add_partial(recv_buf.at[2], False)

        local_amax = jnp.float32(0.0)
        for r in range(0, m_per, STRIP):
            local_amax = jnp.maximum(
                local_amax, jnp.max(jnp.abs(recv_buf[2, pl.ds(r, STRIP), :]))
            )
        amax_send[...] = jnp.full((8, 128), local_amax, jnp.float32)
        descs = []
        for j in (1, 2, 3):
            peer = lax.rem(my + j, N_DEV)
            d = pltpu.make_async_remote_copy(
                src_ref=amax_send,
                dst_ref=amax_recv.at[j - 1],
                send_sem=amax_send_sems.at[j - 1],
                recv_sem=amax_recv_sems.at[j - 1],
                device_id=(peer,),
                device_id_type=_MESH,
            )
            d.start()
            descs.append(d)
        for d in descs:
            d.wait()
        gmax = local_amax
        for j in range(3):
            gmax = jnp.maximum(gmax, amax_recv[j, 0, 0])

        scale = gmax / 127.0
        for r in range(0, m_per, STRIP):
            q = jnp.clip(
                jnp.round(recv_buf[2, pl.ds(r, STRIP), :] / scale),
                -127.0,
                127.0,
            )
            send_buf[pl.ds(r, STRIP), :] = q * scale
        cp = pltpu.make_async_copy(send_buf, out_ref, load_sem)
        cp.start()
        cp.wait()

    return pl.pallas_call(
        body,
        out_shape=jax.ShapeDtypeStruct((m_per, n), jnp.float32),
        in_specs=[
            pl.BlockSpec(memory_space=pltpu.HBM),
            pl.BlockSpec(memory_space=pltpu.VMEM),
        ],
        out_specs=pl.BlockSpec(memory_space=pltpu.HBM),
        scratch_shapes=[
            pltpu.VMEM((m_per, k_per), jnp.float32),
            pltpu.VMEM((m_per, n), jnp.float32),
            pltpu.VMEM((3, m_per, n), jnp.float32),
            pltpu.SemaphoreType.DMA,
            pltpu.SemaphoreType.DMA((3,)),
            pltpu.SemaphoreType.DMA((3,)),
            pltpu.VMEM((8, 128), jnp.float32),
            pltpu.VMEM((3, 8, 128), jnp.float32),
            pltpu.SemaphoreType.DMA((3,)),
            pltpu.SemaphoreType.DMA((3,)),
        ],
        compiler_params=_CompilerParams(
            collective_id=0, vmem_limit_bytes=52 * 1024 * 1024
        ),
    )(x, w_mat)


# device time: 196011 ns/iter; 2.1532x vs baseline; 2.1532x over previous
import jax
import jax.numpy as jnp
from jax import lax
from jax.experimental import pallas as pl
from jax.experimental.pallas import tpu as pltpu

N_DEV = 4

_DID = getattr(pl, "DeviceIdType", None) or getattr(pltpu, "DeviceIdType")
_MESH = _DID.MESH

_CompilerParams = getattr(pltpu, "CompilerParams", None) or getattr(
    pltpu, "TPUCompilerParams"
)


def kernel(x, w_mat):
    m_total, k_per = x.shape
    _, n = w_mat.shape
    m_per = m_total // N_DEV
    nh = n // 2

    def body(
        x_ref,
        w_ref,
        out_ref,
        x_chunk,
        acc_r,
        acc_l,
        recv_r,
        recv_l,
        load_sem,
        out_sems,
        send_sems_r,
        recv_sems_r,
        send_sems_l,
        recv_sems_l,
        credit_r,
        credit_l,
        amax_send,
        amax_recv,
        amax_send_sems,
        amax_recv_sems,
    ):
        my = lax.axis_index("i")
        right = lax.rem(my + 1, N_DEV)
        left = lax.rem(my + N_DEV - 1, N_DEV)

        STRIP = 512

        def load_chunk(c):
            cp = pltpu.make_async_copy(
                x_ref.at[pl.ds(c * m_per, m_per), :], x_chunk, load_sem
            )
            cp.start()
            cp.wait()

        def mm_half(dst, left_half):
            for r in range(0, m_per, STRIP):
                xs = x_chunk[pl.ds(r, STRIP), :]
                ws = w_ref[:, nh:] if left_half else w_ref[:, :nh]
                dst[pl.ds(r, STRIP), :] = jnp.dot(
                    xs,
                    ws,
                    preferred_element_type=jnp.float32,
                    precision=lax.Precision.HIGHEST,
                )

        def accum(dst, src):
            for r in range(0, m_per, STRIP):
                dst[pl.ds(r, STRIP), :] = (
                    dst[pl.ds(r, STRIP), :] + src[pl.ds(r, STRIP), :]
                )

        def hop_rdmas(src_slot, dst_slot, hop):
            rr = pltpu.make_async_remote_copy(
                src_ref=acc_r.at[src_slot],
                dst_ref=recv_r.at[dst_slot],
                send_sem=send_sems_r.at[hop],
                recv_sem=recv_sems_r.at[hop],
                device_id=(right,),
                device_id_type=_MESH,
            )
            rl = pltpu.make_async_remote_copy(
                src_ref=acc_l.at[src_slot],
                dst_ref=recv_l.at[dst_slot],
                send_sem=send_sems_l.at[hop],
                recv_sem=recv_sems_l.at[hop],
                device_id=(left,),
                device_id_type=_MESH,
            )
            return rr, rl

        load_chunk(left)
        mm_half(acc_r.at[0], left_half=False)
        load_chunk(right)
        mm_half(acc_l.at[0], left_half=True)

        barrier = pltpu.get_barrier_semaphore()
        for nbr in (left, right):
            pl.semaphore_signal(
                barrier, inc=1, device_id=(nbr,), device_id_type=_MESH
            )
        pl.semaphore_wait(barrier, 2)

        r0, l0 = hop_rdmas(src_slot=0, dst_slot=0, hop=0)
        r0.start()
        l0.start()

        load_chunk(lax.rem(my + 2, N_DEV))
        mm_half(acc_r.at[1], left_half=False)
        mm_half(acc_l.at[1], left_half=True)

        r0.wait_recv()
        accum(acc_r.at[1], recv_r.at[0])
        pl.semaphore_signal(
            credit_r, inc=1, device_id=(left,), device_id_type=_MESH
        )
        l0.wait_recv()
        accum(acc_l.at[1], recv_l.at[0])
        pl.semaphore_signal(
            credit_l, inc=1, device_id=(right,), device_id_type=_MESH
        )
        r0.wait_send()
        l0.wait_send()

        r1, l1 = hop_rdmas(src_slot=1, dst_slot=1, hop=1)
        r1.start()
        l1.start()

        load_chunk(right)
        mm_half(acc_r.at[0], left_half=False)
        load_chunk(left)
        mm_half(acc_l.at[0], left_half=True)

        r1.wait_recv()
        accum(acc_r.at[0], recv_r.at[1])
        l1.wait_recv()
        accum(acc_l.at[0], recv_l.at[1])
        r1.wait_send()
        l1.wait_send()

        pl.semaphore_wait(credit_r, 1)
        pl.semaphore_wait(credit_l, 1)
        r2, l2 = hop_rdmas(src_slot=0, dst_slot=0, hop=2)
        r2.start()
        l2.start()

        load_chunk(my)
        mm_half(acc_r.at[1], left_half=False)
        mm_half(acc_l.at[1], left_half=True)

        r2.wait_recv()
        accum(recv_r.at[0], acc_r.at[1])
        l2.wait_recv()
        accum(recv_l.at[0], acc_l.at[1])
        r2.wait_send()
        l2.wait_send()

        local_amax = jnp.float32(0.0)
        for r in range(0, m_per, STRIP):
            local_amax = jnp.maximum(
                local_amax,
                jnp.max(jnp.abs(recv_r[0, pl.ds(r, STRIP), :])),
            )
            local_amax = jnp.maximum(
                local_amax,
                jnp.max(jnp.abs(recv_l[0, pl.ds(r, STRIP), :])),
            )
        amax_send[...] = jnp.full((8, 128), local_amax, jnp.float32)
        descs = []
        for j in (1, 2, 3):
            peer = lax.rem(my + j, N_DEV)
            d = pltpu.make_async_remote_copy(
                src_ref=amax_send,
                dst_ref=amax_recv.at[j - 1],
                send_sem=amax_send_sems.at[j - 1],
                recv_sem=amax_recv_sems.at[j - 1],
                device_id=(peer,),
                device_id_type=_MESH,
            )
            d.start()
            descs.append(d)
        for d in descs:
            d.wait()
        gmax = local_amax
        for j in range(3):
            gmax = jnp.maximum(gmax, amax_recv[j, 0, 0])

        scale = gmax / 127.0
        for r in range(0, m_per, STRIP):
            q = jnp.clip(
                jnp.round(recv_r[0, pl.ds(r, STRIP), :] / scale),
                -127.0,
                127.0,
            )
            acc_r[1, pl.ds(r, STRIP), :] = q * scale
            q = jnp.clip(
                jnp.round(recv_l[0, pl.ds(r, STRIP), :] / scale),
                -127.0,
                127.0,
            )
            acc_l[1, pl.ds(r, STRIP), :] = q * scale
        cp_r = pltpu.make_async_copy(
            acc_r.at[1], out_ref.at[:, pl.ds(0, nh)], out_sems.at[0]
        )
        cp_l = pltpu.make_async_copy(
            acc_l.at[1], out_ref.at[:, pl.ds(nh, nh)], out_sems.at[1]
        )
        cp_r.start()
        cp_l.start()
        cp_r.wait()
        cp_l.wait()

    return pl.pallas_call(
        body,
        out_shape=jax.ShapeDtypeStruct((m_per, n), jnp.float32),
        in_specs=[
            pl.BlockSpec(memory_space=pltpu.HBM),
            pl.BlockSpec(memory_space=pltpu.VMEM),
        ],
        out_specs=pl.BlockSpec(memory_space=pltpu.HBM),
        scratch_shapes=[
            pltpu.VMEM((m_per, k_per), jnp.float32),
            pltpu.VMEM((2, m_per, nh), jnp.float32),
            pltpu.VMEM((2, m_per, nh), jnp.float32),
            pltpu.VMEM((2, m_per, nh), jnp.float32),
            pltpu.VMEM((2, m_per, nh), jnp.float32),
            pltpu.SemaphoreType.DMA,
            pltpu.SemaphoreType.DMA((2,)),
            pltpu.SemaphoreType.DMA((3,)),
            pltpu.SemaphoreType.DMA((3,)),
            pltpu.SemaphoreType.DMA((3,)),
            pltpu.SemaphoreType.DMA((3,)),
            pltpu.SemaphoreType.REGULAR,
            pltpu.SemaphoreType.REGULAR,
            pltpu.VMEM((8, 128), jnp.float32),
            pltpu.VMEM((3, 8, 128), jnp.float32),
            pltpu.SemaphoreType.DMA((3,)),
            pltpu.SemaphoreType.DMA((3,)),
        ],
        compiler_params=_CompilerParams(
            collective_id=0, vmem_limit_bytes=54 * 1024 * 1024
        ),
    )(x, w_mat)


# device time: 185142 ns/iter; 2.2797x vs baseline; 1.0587x over previous
import jax
import jax.numpy as jnp
from jax import lax
from jax.experimental import pallas as pl
from jax.experimental.pallas import tpu as pltpu

N_DEV = 4

_DID = getattr(pl, "DeviceIdType", None) or getattr(pltpu, "DeviceIdType")
_MESH = _DID.MESH

_CompilerParams = getattr(pltpu, "CompilerParams", None) or getattr(
    pltpu, "TPUCompilerParams"
)


def kernel(x, w_mat):
    m_total, k_per = x.shape
    _, n = w_mat.shape
    m_per = m_total // N_DEV
    nh = n // 2
    STRIP = 512
    N_SUB = m_per // STRIP

    def body(
        x_ref,
        w_ref,
        out_ref,
        x_chunk,
        acc_r,
        acc_l,
        recv_r,
        recv_l,
        load_sem,
        out_sems,
        send_sems_r,
        recv_sems_r,
        send_sems_l,
        recv_sems_l,
        credit_r,
        credit_l,
        amax_send,
        amax_recv,
        amax_send_sems,
        amax_recv_sems,
    ):
        my = lax.axis_index("i")
        right = lax.rem(my + 1, N_DEV)
        left = lax.rem(my + N_DEV - 1, N_DEV)

        barrier = pltpu.get_barrier_semaphore()
        for nbr in (left, right):
            pl.semaphore_signal(
                barrier, inc=1, device_id=(nbr,), device_id_type=_MESH
            )
        pl.semaphore_wait(barrier, 2)

        def load_chunk(c):
            cp = pltpu.make_async_copy(
                x_ref.at[pl.ds(c * m_per, m_per), :], x_chunk, load_sem
            )
            cp.start()
            cp.wait()

        def mm_sub(acc, slot, sub, left_half):
            ws = w_ref[:, nh:] if left_half else w_ref[:, :nh]
            acc[slot, pl.ds(sub * STRIP, STRIP), :] = jnp.dot(
                x_chunk[pl.ds(sub * STRIP, STRIP), :],
                ws,
                preferred_element_type=jnp.float32,
                precision=lax.Precision.HIGHEST,
            )

        def accum_sub(acc, aslot, rbuf, rslot, sub):
            rs = pl.ds(sub * STRIP, STRIP)
            acc[aslot, rs, :] = acc[aslot, rs, :] + rbuf[rslot, rs, :]

        def sub_rdma(dirn, src_slot, dst_slot, hop, sub):
            acc, rbuf, ss, rs, dev = (
                (acc_r, recv_r, send_sems_r, recv_sems_r, right)
                if dirn == 0
                else (acc_l, recv_l, send_sems_l, recv_sems_l, left)
            )
            rows = pl.ds(sub * STRIP, STRIP)
            return pltpu.make_async_remote_copy(
                src_ref=acc.at[src_slot, rows, :],
                dst_ref=rbuf.at[dst_slot, rows, :],
                send_sem=ss.at[hop * N_SUB + sub],
                recv_sem=rs.at[hop * N_SUB + sub],
                device_id=(dev,),
                device_id_type=_MESH,
            )

        load_chunk(left)
        mm_sub(acc_r, 0, 0, False)
        r0a = sub_rdma(0, 0, 0, 0, 0)
        r0a.start()
        mm_sub(acc_r, 0, 1, False)
        r0b = sub_rdma(0, 0, 0, 0, 1)
        r0b.start()
        load_chunk(right)
        mm_sub(acc_l, 0, 0, True)
        l0a = sub_rdma(1, 0, 0, 0, 0)
        l0a.start()
        mm_sub(acc_l, 0, 1, True)
        l0b = sub_rdma(1, 0, 0, 0, 1)
        l0b.start()

        load_chunk(lax.rem(my + 2, N_DEV))
        mm_sub(acc_r, 1, 0, False)
        mm_sub(acc_r, 1, 1, False)
        mm_sub(acc_l, 1, 0, True)
        mm_sub(acc_l, 1, 1, True)

        r0a.wait_recv()
        accum_sub(acc_r, 1, recv_r, 0, 0)
        r1a = sub_rdma(0, 1, 1, 1, 0)
        r1a.start()
        l0a.wait_recv()
        accum_sub(acc_l, 1, recv_l, 0, 0)
        l1a = sub_rdma(1, 1, 1, 1, 0)
        l1a.start()
        r0b.wait_recv()
        accum_sub(acc_r, 1, recv_r, 0, 1)
        r1b = sub_rdma(0, 1, 1, 1, 1)
        r1b.start()
        l0b.wait_recv()
        accum_sub(acc_l, 1, recv_l, 0, 1)
        l1b = sub_rdma(1, 1, 1, 1, 1)
        l1b.start()
        pl.semaphore_signal(
            credit_r, inc=1, device_id=(left,), device_id_type=_MESH
        )
        pl.semaphore_signal(
            credit_l, inc=1, device_id=(right,), device_id_type=_MESH
        )

        r0a.wait_send()
        r0b.wait_send()
        l0a.wait_send()
        l0b.wait_send()
        load_chunk(right)
        mm_sub(acc_r, 0, 0, False)
        mm_sub(acc_r, 0, 1, False)
        load_chunk(left)
        mm_sub(acc_l, 0, 0, True)
        mm_sub(acc_l, 0, 1, True)

        pl.semaphore_wait(credit_r, 1)
        pl.semaphore_wait(credit_l, 1)
        r1a.wait_recv()
        accum_sub(acc_r, 0, recv_r, 1, 0)
        r2a = sub_rdma(0, 0, 0, 2, 0)
        r2a.start()
        l1a.wait_recv()
        accum_sub(acc_l, 0, recv_l, 1, 0)
        l2a = sub_rdma(1, 0, 0, 2, 0)
        l2a.start()
        r1b.wait_recv()
        accum_sub(acc_r, 0, recv_r, 1, 1)
        r2b = sub_rdma(0, 0, 0, 2, 1)
        r2b.start()
        l1b.wait_recv()
        accum_sub(acc_l, 0, recv_l, 1, 1)
        l2b = sub_rdma(1, 0, 0, 2, 1)
        l2b.start()

        r1a.wait_send()
        r1b.wait_send()
        l1a.wait_send()
        l1b.wait_send()
        load_chunk(my)
        mm_sub(acc_r, 1, 0, False)
        mm_sub(acc_r, 1, 1, False)
        mm_sub(acc_l, 1, 0, True)
        mm_sub(acc_l, 1, 1, True)

        def accum_final_sub(rbuf, acc, sub):
            rs = pl.ds(sub * STRIP, STRIP)
            v = rbuf[0, rs, :] + acc[1, rs, :]
            rbuf[0, rs, :] = v
            return jnp.max(jnp.abs(v))

        r2a.wait_recv()
        local_amax = accum_final_sub(recv_r, acc_r, 0)
        l2a.wait_recv()
        local_amax = jnp.maximum(local_amax, accum_final_sub(recv_l, acc_l, 0))
        r2b.wait_recv()
        local_amax = jnp.maximum(local_amax, accum_final_sub(recv_r, acc_r, 1))
        l2b.wait_recv()
        local_amax = jnp.maximum(local_amax, accum_final_sub(recv_l, acc_l, 1))
        r2a.wait_send()
        r2b.wait_send()
        l2a.wait_send()
        l2b.wait_send()

        amax_send[...] = jnp.full((8, 128), local_amax, jnp.float32)
        descs = []
        for j in (1, 2, 3):
            peer = lax.rem(my + j, N_DEV)
            d = pltpu.make_async_remote_copy(
                src_ref=amax_send,
                dst_ref=amax_recv.at[j - 1],
                send_sem=amax_send_sems.at[j - 1],
                recv_sem=amax_recv_sems.at[j - 1],
                device_id=(peer,),
                device_id_type=_MESH,
            )
            d.start()
            descs.append(d)
        for d in descs:
            d.wait()
        gmax = local_amax
        for j in range(3):
            gmax = jnp.maximum(gmax, amax_recv[j, 0, 0])

        scale = gmax / 127.0
        inv_scale = 127.0 / gmax
        for sub in range(N_SUB):
            rs = pl.ds(sub * STRIP, STRIP)
            q = jnp.clip(jnp.round(recv_r[0, rs, :] * inv_scale), -127.0, 127.0)
            acc_r[1, rs, :] = q * scale
        cp_r = pltpu.make_async_copy(
            acc_r.at[1], out_ref.at[:, pl.ds(0, nh)], out_sems.at[0]
        )
        cp_r.start()
        for sub in range(N_SUB):
            rs = pl.ds(sub * STRIP, STRIP)
            q = jnp.clip(jnp.round(recv_l[0, rs, :] * inv_scale), -127.0, 127.0)
            acc_l[1, rs, :] = q * scale
        cp_l = pltpu.make_async_copy(
            acc_l.at[1], out_ref.at[:, pl.ds(nh, nh)], out_sems.at[1]
        )
        cp_l.start()
        cp_r.wait()
        cp_l.wait()

    return pl.pallas_call(
        body,
        out_shape=jax.ShapeDtypeStruct((m_per, n), jnp.float32),
        in_specs=[
            pl.BlockSpec(memory_space=pltpu.HBM),
            pl.BlockSpec(memory_space=pltpu.VMEM),
        ],
        out_specs=pl.BlockSpec(memory_space=pltpu.HBM),
        scratch_shapes=[
            pltpu.VMEM((m_per, k_per), jnp.float32),
            pltpu.VMEM((2, m_per, nh), jnp.float32),
            pltpu.VMEM((2, m_per, nh), jnp.float32),
            pltpu.VMEM((2, m_per, nh), jnp.float32),
            pltpu.VMEM((2, m_per, nh), jnp.float32),
            pltpu.SemaphoreType.DMA,
            pltpu.SemaphoreType.DMA((2,)),
            pltpu.SemaphoreType.DMA((6,)),
            pltpu.SemaphoreType.DMA((6,)),
            pltpu.SemaphoreType.DMA((6,)),
            pltpu.SemaphoreType.DMA((6,)),
            pltpu.SemaphoreType.REGULAR,
            pltpu.SemaphoreType.REGULAR,
            pltpu.VMEM((8, 128), jnp.float32),
            pltpu.VMEM((3, 8, 128), jnp.float32),
            pltpu.SemaphoreType.DMA((3,)),
            pltpu.SemaphoreType.DMA((3,)),
        ],
        compiler_params=_CompilerParams(
            collective_id=0, vmem_limit_bytes=54 * 1024 * 1024
        ),
    )(x, w_mat)


# device time: 165769 ns/iter; 2.5461x vs baseline; 1.1169x over previous
import jax
import jax.numpy as jnp
from jax import lax
from jax.experimental import pallas as pl
from jax.experimental.pallas import tpu as pltpu

N_DEV = 4

_DID = getattr(pl, "DeviceIdType", None) or getattr(pltpu, "DeviceIdType")
_MESH = _DID.MESH

_CompilerParams = getattr(pltpu, "CompilerParams", None) or getattr(
    pltpu, "TPUCompilerParams"
)


def kernel(x, w_mat):
    m_total, k_per = x.shape
    _, n = w_mat.shape
    m_per = m_total // N_DEV
    nh = n // 2
    STRIP = 512
    N_SUB = m_per // STRIP

    def body(
        x_ref,
        w_ref,
        out_ref,
        x_chunk,
        acc_r,
        acc_l,
        recv_r,
        recv_l,
        load_sem,
        out_sems,
        send_sems_r,
        recv_sems_r,
        send_sems_l,
        recv_sems_l,
        credit_r,
        credit_l,
        amax_send,
        amax_recv,
        amax_send_sems,
        amax_recv_sems,
    ):
        my = lax.axis_index("i")
        right = lax.rem(my + 1, N_DEV)
        left = lax.rem(my + N_DEV - 1, N_DEV)

        barrier = pltpu.get_barrier_semaphore()
        for nbr in (left, right):
            pl.semaphore_signal(
                barrier, inc=1, device_id=(nbr,), device_id_type=_MESH
            )
        pl.semaphore_wait(barrier, 2)

        def load_chunk(c):
            cp = pltpu.make_async_copy(
                x_ref.at[pl.ds(c * m_per, m_per), :], x_chunk, load_sem
            )
            cp.start()
            cp.wait()

        def mm_sub(acc, slot, sub, left_half):
            ws = w_ref[:, nh:] if left_half else w_ref[:, :nh]
            acc[slot, pl.ds(sub * STRIP, STRIP), :] = jnp.dot(
                x_chunk[pl.ds(sub * STRIP, STRIP), :],
                ws,
                preferred_element_type=jnp.float32,
                precision=lax.Precision.DEFAULT,
            )

        def accum_sub(acc, aslot, rbuf, rslot, sub):
            rs = pl.ds(sub * STRIP, STRIP)
            acc[aslot, rs, :] = acc[aslot, rs, :] + rbuf[rslot, rs, :]

        def sub_rdma(dirn, src_slot, dst_slot, hop, sub):
            acc, rbuf, ss, rs, dev = (
                (acc_r, recv_r, send_sems_r, recv_sems_r, right)
                if dirn == 0
                else (acc_l, recv_l, send_sems_l, recv_sems_l, left)
            )
            rows = pl.ds(sub * STRIP, STRIP)
            return pltpu.make_async_remote_copy(
                src_ref=acc.at[src_slot, rows, :],
                dst_ref=rbuf.at[dst_slot, rows, :],
                send_sem=ss.at[hop * N_SUB + sub],
                recv_sem=rs.at[hop * N_SUB + sub],
                device_id=(dev,),
                device_id_type=_MESH,
            )

        load_chunk(left)
        mm_sub(acc_r, 0, 0, False)
        r0a = sub_rdma(0, 0, 0, 0, 0)
        r0a.start()
        mm_sub(acc_r, 0, 1, False)
        r0b = sub_rdma(0, 0, 0, 0, 1)
        r0b.start()
        load_chunk(right)
        mm_sub(acc_l, 0, 0, True)
        l0a = sub_rdma(1, 0, 0, 0, 0)
        l0a.start()
        mm_sub(acc_l, 0, 1, True)
        l0b = sub_rdma(1, 0, 0, 0, 1)
        l0b.start()

        load_chunk(lax.rem(my + 2, N_DEV))
        mm_sub(acc_r, 1, 0, False)
        mm_sub(acc_r, 1, 1, False)
        mm_sub(acc_l, 1, 0, True)
        mm_sub(acc_l, 1, 1, True)

        r0a.wait_recv()
        accum_sub(acc_r, 1, recv_r, 0, 0)
        r1a = sub_rdma(0, 1, 1, 1, 0)
        r1a.start()
        l0a.wait_recv()
        accum_sub(acc_l, 1, recv_l, 0, 0)
        l1a = sub_rdma(1, 1, 1, 1, 0)
        l1a.start()
        r0b.wait_recv()
        accum_sub(acc_r, 1, recv_r, 0, 1)
        r1b = sub_rdma(0, 1, 1, 1, 1)
        r1b.start()
        l0b.wait_recv()
        accum_sub(acc_l, 1, recv_l, 0, 1)
        l1b = sub_rdma(1, 1, 1, 1, 1)
        l1b.start()
        pl.semaphore_signal(
            credit_r, inc=1, device_id=(left,), device_id_type=_MESH
        )
        pl.semaphore_signal(
            credit_l, inc=1, device_id=(right,), device_id_type=_MESH
        )

        r0a.wait_send()
        r0b.wait_send()
        l0a.wait_send()
        l0b.wait_send()
        load_chunk(right)
        mm_sub(acc_r, 0, 0, False)
        mm_sub(acc_r, 0, 1, False)
        load_chunk(left)
        mm_sub(acc_l, 0, 0, True)
        mm_sub(acc_l, 0, 1, True)

        pl.semaphore_wait(credit_r, 1)
        pl.semaphore_wait(credit_l, 1)
        r1a.wait_recv()
        accum_sub(acc_r, 0, recv_r, 1, 0)
        r2a = sub_rdma(0, 0, 0, 2, 0)
        r2a.start()
        l1a.wait_recv()
        accum_sub(acc_l, 0, recv_l, 1, 0)
        l2a = sub_rdma(1, 0, 0, 2, 0)
        l2a.start()
        r1b.wait_recv()
        accum_sub(acc_r, 0, recv_r, 1, 1)
        r2b = sub_rdma(0, 0, 0, 2, 1)
        r2b.start()
        l1b.wait_recv()
        accum_sub(acc_l, 0, recv_l, 1, 1)
        l2b = sub_rdma(1, 0, 0, 2, 1)
        l2b.start()

        r1a.wait_send()
        r1b.wait_send()
        l1a.wait_send()
        l1b.wait_send()
        load_chunk(my)
        mm_sub(acc_r, 1, 0, False)
        mm_sub(acc_r, 1, 1, False)
        mm_sub(acc_l, 1, 0, True)
        mm_sub(acc_l, 1, 1, True)

        def accum_final_sub(rbuf, acc, sub):
            rs = pl.ds(sub * STRIP, STRIP)
            v = rbuf[0, rs, :] + acc[1, rs, :]
            rbuf[0, rs, :] = v
            return jnp.max(jnp.abs(v))

        r2a.wait_recv()
        local_amax = accum_final_sub(recv_r, acc_r, 0)
        l2a.wait_recv()
        local_amax = jnp.maximum(local_amax, accum_final_sub(recv_l, acc_l, 0))
        r2b.wait_recv()
        local_amax = jnp.maximum(local_amax, accum_final_sub(recv_r, acc_r, 1))
        l2b.wait_recv()
        local_amax = jnp.maximum(local_amax, accum_final_sub(recv_l, acc_l, 1))
        r2a.wait_send()
        r2b.wait_send()
        l2a.wait_send()
        l2b.wait_send()

        amax_send[...] = jnp.full((8, 128), local_amax, jnp.float32)
        descs = []
        for j in (1, 2, 3):
            peer = lax.rem(my + j, N_DEV)
            d = pltpu.make_async_remote_copy(
                src_ref=amax_send,
                dst_ref=amax_recv.at[j - 1],
                send_sem=amax_send_sems.at[j - 1],
                recv_sem=amax_recv_sems.at[j - 1],
                device_id=(peer,),
                device_id_type=_MESH,
            )
            d.start()
            descs.append(d)
        for d in descs:
            d.wait()
        gmax = local_amax
        for j in range(3):
            gmax = jnp.maximum(gmax, amax_recv[j, 0, 0])

        scale = gmax / 127.0
        inv_scale = 127.0 / gmax
        for sub in range(N_SUB):
            rs = pl.ds(sub * STRIP, STRIP)
            q = jnp.clip(jnp.round(recv_r[0, rs, :] * inv_scale), -127.0, 127.0)
            acc_r[1, rs, :] = q * scale
        cp_r = pltpu.make_async_copy(
            acc_r.at[1], out_ref.at[:, pl.ds(0, nh)], out_sems.at[0]
        )
        cp_r.start()
        for sub in range(N_SUB):
            rs = pl.ds(sub * STRIP, STRIP)
            q = jnp.clip(jnp.round(recv_l[0, rs, :] * inv_scale), -127.0, 127.0)
            acc_l[1, rs, :] = q * scale
        cp_l = pltpu.make_async_copy(
            acc_l.at[1], out_ref.at[:, pl.ds(nh, nh)], out_sems.at[1]
        )
        cp_l.start()
        cp_r.wait()
        cp_l.wait()

    return pl.pallas_call(
        body,
        out_shape=jax.ShapeDtypeStruct((m_per, n), jnp.float32),
        in_specs=[
            pl.BlockSpec(memory_space=pltpu.HBM),
            pl.BlockSpec(memory_space=pltpu.VMEM),
        ],
        out_specs=pl.BlockSpec(memory_space=pltpu.HBM),
        scratch_shapes=[
            pltpu.VMEM((m_per, k_per), jnp.float32),
            pltpu.VMEM((2, m_per, nh), jnp.float32),
            pltpu.VMEM((2, m_per, nh), jnp.float32),
            pltpu.VMEM((2, m_per, nh), jnp.float32),
            pltpu.VMEM((2, m_per, nh), jnp.float32),
            pltpu.SemaphoreType.DMA,
            pltpu.SemaphoreType.DMA((2,)),
            pltpu.SemaphoreType.DMA((6,)),
            pltpu.SemaphoreType.DMA((6,)),
            pltpu.SemaphoreType.DMA((6,)),
            pltpu.SemaphoreType.DMA((6,)),
            pltpu.SemaphoreType.REGULAR,
            pltpu.SemaphoreType.REGULAR,
            pltpu.VMEM((8, 128), jnp.float32),
            pltpu.VMEM((3, 8, 128), jnp.float32),
            pltpu.SemaphoreType.DMA((3,)),
            pltpu.SemaphoreType.DMA((3,)),
        ],
        compiler_params=_CompilerParams(
            collective_id=0, vmem_limit_bytes=54 * 1024 * 1024
        ),
    )(x, w_mat)


# device time: 98421 ns/iter; 4.2883x vs baseline; 1.6843x over previous
import jax
import jax.numpy as jnp
from jax import lax
from jax.experimental import pallas as pl
from jax.experimental.pallas import tpu as pltpu

N_DEV = 4

_DID = getattr(pl, "DeviceIdType", None) or getattr(pltpu, "DeviceIdType")
_MESH = _DID.MESH

_CompilerParams = getattr(pltpu, "CompilerParams", None) or getattr(
    pltpu, "TPUCompilerParams"
)


def kernel(x, w_mat):
    m_total, k_per = x.shape
    _, n = w_mat.shape
    m_per = m_total // N_DEV
    nh = n // 2
    STRIP = 512
    N_SUB = m_per // STRIP

    def body(
        x_ref,
        w_ref,
        out_ref,
        x_chunk,
        acc_r,
        acc_l,
        recv_r,
        recv_l,
        y_final_r,
        y_final_l,
        load_sem,
        out_sems,
        send_sems_r,
        recv_sems_r,
        send_sems_l,
        recv_sems_l,
        credit_r,
        credit_l,
        amax_send,
        amax_recv,
        amax_send_sems,
        amax_recv_sems,
    ):
        my = lax.axis_index("i")
        right = lax.rem(my + 1, N_DEV)
        left = lax.rem(my + N_DEV - 1, N_DEV)

        barrier = pltpu.get_barrier_semaphore()
        for nbr in (left, right):
            pl.semaphore_signal(
                barrier, inc=1, device_id=(nbr,), device_id_type=_MESH
            )
        pl.semaphore_wait(barrier, 2)

        def load_chunk(c):
            cp = pltpu.make_async_copy(
                x_ref.at[pl.ds(c * m_per, m_per), :], x_chunk, load_sem
            )
            cp.start()
            cp.wait()

        def mm_sub(acc, slot, sub, left_half):
            ws = w_ref[:, nh:] if left_half else w_ref[:, :nh]
            acc[slot, pl.ds(sub * STRIP, STRIP), :] = jnp.dot(
                x_chunk[pl.ds(sub * STRIP, STRIP), :],
                ws,
                preferred_element_type=jnp.float32,
                precision=lax.Precision.DEFAULT,
            ).astype(jnp.bfloat16)

        def accum_sub(acc, aslot, rbuf, rslot, sub):
            rs = pl.ds(sub * STRIP, STRIP)
            acc[aslot, rs, :] = (
                acc[aslot, rs, :].astype(jnp.float32)
                + rbuf[rslot, rs, :].astype(jnp.float32)
            ).astype(jnp.bfloat16)

        def sub_rdma(dirn, src_slot, dst_slot, hop, sub):
            acc, rbuf, ss, rs, dev = (
                (acc_r, recv_r, send_sems_r, recv_sems_r, right)
                if dirn == 0
                else (acc_l, recv_l, send_sems_l, recv_sems_l, left)
            )
            rows = pl.ds(sub * STRIP, STRIP)
            return pltpu.make_async_remote_copy(
                src_ref=acc.at[src_slot, rows, :],
                dst_ref=rbuf.at[dst_slot, rows, :],
                send_sem=ss.at[hop * N_SUB + sub],
                recv_sem=rs.at[hop * N_SUB + sub],
                device_id=(dev,),
                device_id_type=_MESH,
            )

        load_chunk(left)
        mm_sub(acc_r, 0, 0, False)
        r0a = sub_rdma(0, 0, 0, 0, 0)
        r0a.start()
        mm_sub(acc_r, 0, 1, False)
        r0b = sub_rdma(0, 0, 0, 0, 1)
        r0b.start()
        load_chunk(right)
        mm_sub(acc_l, 0, 0, True)
        l0a = sub_rdma(1, 0, 0, 0, 0)
        l0a.start()
        mm_sub(acc_l, 0, 1, True)
        l0b = sub_rdma(1, 0, 0, 0, 1)
        l0b.start()

        load_chunk(lax.rem(my + 2, N_DEV))
        mm_sub(acc_r, 1, 0, False)
        mm_sub(acc_r, 1, 1, False)
        mm_sub(acc_l, 1, 0, True)
        mm_sub(acc_l, 1, 1, True)

        r0a.wait_recv()
        accum_sub(acc_r, 1, recv_r, 0, 0)
        r1a = sub_rdma(0, 1, 1, 1, 0)
        r1a.start()
        l0a.wait_recv()
        accum_sub(acc_l, 1, recv_l, 0, 0)
        l1a = sub_rdma(1, 1, 1, 1, 0)
        l1a.start()
        r0b.wait_recv()
        accum_sub(acc_r, 1, recv_r, 0, 1)
        r1b = sub_rdma(0, 1, 1, 1, 1)
        r1b.start()
        l0b.wait_recv()
        accum_sub(acc_l, 1, recv_l, 0, 1)
        l1b = sub_rdma(1, 1, 1, 1, 1)
        l1b.start()
        pl.semaphore_signal(
            credit_r, inc=1, device_id=(left,), device_id_type=_MESH
        )
        pl.semaphore_signal(
            credit_l, inc=1, device_id=(right,), device_id_type=_MESH
        )

        r0a.wait_send()
        r0b.wait_send()
        l0a.wait_send()
        l0b.wait_send()
        load_chunk(right)
        mm_sub(acc_r, 0, 0, False)
        mm_sub(acc_r, 0, 1, False)
        load_chunk(left)
        mm_sub(acc_l, 0, 0, True)
        mm_sub(acc_l, 0, 1, True)

        pl.semaphore_wait(credit_r, 1)
        pl.semaphore_wait(credit_l, 1)
        r1a.wait_recv()
        accum_sub(acc_r, 0, recv_r, 1, 0)
        r2a = sub_rdma(0, 0, 0, 2, 0)
        r2a.start()
        l1a.wait_recv()
        accum_sub(acc_l, 0, recv_l, 1, 0)
        l2a = sub_rdma(1, 0, 0, 2, 0)
        l2a.start()
        r1b.wait_recv()
        accum_sub(acc_r, 0, recv_r, 1, 1)
        r2b = sub_rdma(0, 0, 0, 2, 1)
        r2b.start()
        l1b.wait_recv()
        accum_sub(acc_l, 0, recv_l, 1, 1)
        l2b = sub_rdma(1, 0, 0, 2, 1)
        l2b.start()

        r1a.wait_send()
        r1b.wait_send()
        l1a.wait_send()
        l1b.wait_send()
        load_chunk(my)
        for sub in range(N_SUB):
            rs = pl.ds(sub * STRIP, STRIP)
            xs = x_chunk[rs, :]
            y_final_r[rs, :] = jnp.dot(
                xs, w_ref[:, :nh], preferred_element_type=jnp.float32
            )
            y_final_l[rs, :] = jnp.dot(
                xs, w_ref[:, nh:], preferred_element_type=jnp.float32
            )

        def accum_final_sub(yf, rbuf, sub):
            rs = pl.ds(sub * STRIP, STRIP)
            v = yf[rs, :] + rbuf[0, rs, :].astype(jnp.float32)
            yf[rs, :] = v
            return jnp.max(jnp.abs(v))

        r2a.wait_recv()
        local_amax = accum_final_sub(y_final_r, recv_r, 0)
        l2a.wait_recv()
        local_amax = jnp.maximum(local_amax, accum_final_sub(y_final_l, recv_l, 0))
        r2b.wait_recv()
        local_amax = jnp.maximum(local_amax, accum_final_sub(y_final_r, recv_r, 1))
        l2b.wait_recv()
        local_amax = jnp.maximum(local_amax, accum_final_sub(y_final_l, recv_l, 1))
        r2a.wait_send()
        r2b.wait_send()
        l2a.wait_send()
        l2b.wait_send()

        amax_send[...] = jnp.full((8, 128), local_amax, jnp.float32)
        descs = []
        for j in (1, 2, 3):
            peer = lax.rem(my + j, N_DEV)
            d = pltpu.make_async_remote_copy(
                src_ref=amax_send,
                dst_ref=amax_recv.at[j - 1],
                send_sem=amax_send_sems.at[j - 1],
                recv_sem=amax_recv_sems.at[j - 1],
                device_id=(peer,),
                device_id_type=_MESH,
            )
            d.start()
            descs.append(d)
        for d in descs:
            d.wait()
        gmax = local_amax
        for j in range(3):
            gmax = jnp.maximum(gmax, amax_recv[j, 0, 0])

        scale = gmax / 127.0
        inv_scale = 127.0 / gmax
        for sub in range(N_SUB):
            rs = pl.ds(sub * STRIP, STRIP)
            q = jnp.clip(jnp.round(y_final_r[rs, :] * inv_scale), -127.0, 127.0)
            y_final_r[rs, :] = q * scale
        cp_r = pltpu.make_async_copy(
            y_final_r, out_ref.at[:, pl.ds(0, nh)], out_sems.at[0]
        )
        cp_r.start()
        for sub in range(N_SUB):
            rs = pl.ds(sub * STRIP, STRIP)
            q = jnp.clip(jnp.round(y_final_l[rs, :] * inv_scale), -127.0, 127.0)
            y_final_l[rs, :] = q * scale
        cp_l = pltpu.make_async_copy(
            y_final_l, out_ref.at[:, pl.ds(nh, nh)], out_sems.at[1]
        )
        cp_l.start()
        cp_r.wait()
        cp_l.wait()

    return pl.pallas_call(
        body,
        out_shape=jax.ShapeDtypeStruct((m_per, n), jnp.float32),
        in_specs=[
            pl.BlockSpec(memory_space=pltpu.HBM),
            pl.BlockSpec(memory_space=pltpu.VMEM),
        ],
        out_specs=pl.BlockSpec(memory_space=pltpu.HBM),
        scratch_shapes=[
            pltpu.VMEM((m_per, k_per), jnp.float32),
            pltpu.VMEM((2, m_per, nh), jnp.bfloat16),
            pltpu.VMEM((2, m_per, nh), jnp.bfloat16),
            pltpu.VMEM((2, m_per, nh), jnp.bfloat16),
            pltpu.VMEM((2, m_per, nh), jnp.bfloat16),
            pltpu.VMEM((m_per, nh), jnp.float32),
            pltpu.VMEM((m_per, nh), jnp.float32),
            pltpu.SemaphoreType.DMA,
            pltpu.SemaphoreType.DMA((2,)),
            pltpu.SemaphoreType.DMA((6,)),
            pltpu.SemaphoreType.DMA((6,)),
            pltpu.SemaphoreType.DMA((6,)),
            pltpu.SemaphoreType.DMA((6,)),
            pltpu.SemaphoreType.REGULAR,
            pltpu.SemaphoreType.REGULAR,
            pltpu.VMEM((8, 128), jnp.float32),
            pltpu.VMEM((3, 8, 128), jnp.float32),
            pltpu.SemaphoreType.DMA((3,)),
            pltpu.SemaphoreType.DMA((3,)),
        ],
        compiler_params=_CompilerParams(
            collective_id=0, vmem_limit_bytes=54 * 1024 * 1024
        ),
    )(x, w_mat)
